# Initial kernel scaffold; baseline (speedup 1.0000x reference)
import functools

import jax
import jax.numpy as jnp
import numpy as np
from jax import lax
from jax.experimental import pallas as pl
from jax.experimental.pallas import tpu as pltpu
from jax.experimental.pallas import tpu_sc as plsc

BOUND = 1.0
NUM_LEVELS = 16
LEVEL_DIM = 2
BASE_RES = 16
LOG2_HASH = 19
MAX_RES = 2048
W_MLP = 64
N_GEO = 15
OUT_CH = 1 + N_GEO
PRIME1 = np.int32(np.int64(2654435761) - (1 << 32))
PRIME2 = np.int32(805459861)
HASH_MASK = np.int32((1 << LOG2_HASH) - 1)


def _levels():
    g = np.exp((np.log(MAX_RES) - np.log(BASE_RES)) / (NUM_LEVELS - 1))
    out, off = [], 0
    for l in range(NUM_LEVELS):
        res = int(np.floor(BASE_RES * (g**l)))
        size = min((res + 1) ** 3, 2**LOG2_HASH)
        size = int(np.ceil(size / 8) * 8)
        dense = (res + 1) ** 3 <= size
        out.append(dict(res=res, size=size, off=off, dense=dense))
        off += size
    return out, off


LEVELS, TOTAL_ROWS = _levels()

NUM_CORES = 2
NUM_SUBCORES = 16
NW = NUM_CORES * NUM_SUBCORES
LANES = 16

C = 512
NBLK = C // LANES
NSEG = (8 * C) // 128


def _sc_encode_body(xs, ys, zs, table, enc_hbm,
                    x01x, x01y, x01z, fx, fy, fz, idx_v, vals_v, enc_v, sem,
                    n_points):
    pw = n_points // NW
    nchunks = pw // C
    wid = lax.axis_index("s") * NUM_CORES + lax.axis_index("c")
    wbase = wid * pw
    lanes = lax.iota(jnp.int32, LANES)
    zeros_i = jnp.zeros((LANES,), jnp.int32)
    ones_i = jnp.ones((LANES,), jnp.int32)

    def chunk_body(k, _):
        pb = wbase + k * C
        pltpu.sync_copy(xs.at[pl.ds(pb, C)], x01x)
        pltpu.sync_copy(ys.at[pl.ds(pb, C)], x01y)
        pltpu.sync_copy(zs.at[pl.ds(pb, C)], x01z)

        def norm_body(i, _):
            i0 = i * LANES
            for ref in (x01x, x01y, x01z):
                v = ref[pl.ds(i0, LANES)]
                v = v * (0.5 / BOUND) + 0.5
                v = jnp.minimum(jnp.maximum(v, 0.0), 1.0)
                ref[pl.ds(i0, LANES)] = v
            return 0

        lax.fori_loop(0, NBLK, norm_body, 0)

        for li, lv in enumerate(LEVELS):
            scale = np.float32(lv["res"] - 1)
            R = np.int32(lv["res"] + 1)
            off = np.int32(lv["off"])

            def passA(i, _, scale=scale, R=R, off=off, dense=lv["dense"]):
                i0 = i * LANES
                px = x01x[pl.ds(i0, LANES)] * scale
                py = x01y[pl.ds(i0, LANES)] * scale
                pz = x01z[pl.ds(i0, LANES)] * scale
                ix0 = px.astype(jnp.int32)
                iy0 = py.astype(jnp.int32)
                iz0 = pz.astype(jnp.int32)
                fx[pl.ds(i0, LANES)] = px - ix0.astype(jnp.float32)
                fy[pl.ds(i0, LANES)] = py - iy0.astype(jnp.float32)
                fz[pl.ds(i0, LANES)] = pz - iz0.astype(jnp.float32)
                ix1 = ix0 + 1
                iy1 = iy0 + 1
                iz1 = iz0 + 1
                if dense:
                    ya = iy0 * R
                    yb = iy1 * R
                    za = iz0 * (R * R) + off
                    zb = iz1 * (R * R) + off
                    for c in range(8):
                        xi = ix1 if (c & 1) else ix0
                        yi = yb if (c & 2) else ya
                        zi = zb if (c & 4) else za
                        idx_v[pl.ds(c * C + i0, LANES)] = xi + yi + zi
                else:
                    ya = iy0 * PRIME1
                    yb = iy1 * PRIME1
                    za = iz0 * PRIME2
                    zb = iz1 * PRIME2
                    for c in range(8):
                        xi = ix1 if (c & 1) else ix0
                        yi = yb if (c & 2) else ya
                        zi = zb if (c & 4) else za
                        h = (xi ^ yi ^ zi) & HASH_MASK
                        idx_v[pl.ds(c * C + i0, LANES)] = h + off
                return 0

            lax.fori_loop(0, NBLK, passA, 0)

            def fire(j, _):
                pltpu.make_async_copy(
                    table.at[idx_v.at[pl.ds(j * 128, 128)]],
                    vals_v.at[pl.ds(j * 128, 128)],
                    sem,
                ).start()
                return 0

            lax.fori_loop(0, NSEG, fire, 0)

            def drain(j, _):
                pltpu.make_async_copy(
                    table.at[idx_v.at[pl.ds(j * 128, 128)]],
                    vals_v.at[pl.ds(j * 128, 128)],
                    sem,
                ).wait()
                return 0

            lax.fori_loop(0, NSEG, drain, 0)

            def passB(i, _, li=li):
                i0 = i * LANES
                fxv = fx[pl.ds(i0, LANES)]
                fyv = fy[pl.ds(i0, LANES)]
                fzv = fz[pl.ds(i0, LANES)]
                gx = 1.0 - fxv
                gy = 1.0 - fyv
                gz = 1.0 - fzv
                rowb = i0 + lanes
                acc0 = jnp.zeros((LANES,), jnp.float32)
                acc1 = jnp.zeros((LANES,), jnp.float32)
                for c in range(8):
                    wx = fxv if (c & 1) else gx
                    wy = fyv if (c & 2) else gy
                    wz = fzv if (c & 4) else gz
                    w = (wx * wy) * wz
                    rows = rowb + np.int32(c * C)
                    v0 = plsc.load_gather(vals_v, [rows, zeros_i])
                    v1 = plsc.load_gather(vals_v, [rows, ones_i])
                    acc0 = acc0 + v0 * w
                    acc1 = acc1 + v1 * w
                enc_v[2 * li, pl.ds(i0, LANES)] = acc0
                enc_v[2 * li + 1, pl.ds(i0, LANES)] = acc1
                return 0

            lax.fori_loop(0, NBLK, passB, 0)

        pltpu.sync_copy(enc_v, enc_hbm.at[:, pl.ds(pb, C)])
        return 0

    lax.fori_loop(0, nchunks, chunk_body, 0)


@functools.partial(jax.jit, static_argnames=("n_points",))
def _sc_encode(xs, ys, zs, table, n_points):
    mesh = plsc.VectorSubcoreMesh(
        core_axis_name="c", subcore_axis_name="s",
        num_cores=NUM_CORES, num_subcores=NUM_SUBCORES)
    body = functools.partial(_sc_encode_body, n_points=n_points)
    return pl.kernel(
        body,
        out_type=jax.ShapeDtypeStruct((NUM_LEVELS * LEVEL_DIM, n_points),
                                      jnp.float32),
        mesh=mesh,
        scratch_types=[
            pltpu.VMEM((C,), jnp.float32),
            pltpu.VMEM((C,), jnp.float32),
            pltpu.VMEM((C,), jnp.float32),
            pltpu.VMEM((C,), jnp.float32),
            pltpu.VMEM((C,), jnp.float32),
            pltpu.VMEM((C,), jnp.float32),
            pltpu.VMEM((8 * C,), jnp.int32),
            pltpu.VMEM((8 * C, LEVEL_DIM), jnp.float32),
            pltpu.VMEM((NUM_LEVELS * LEVEL_DIM, C), jnp.float32),
            pltpu.SemaphoreType.DMA,
        ],
    )(xs, ys, zs, table)


BM = 4096


def _mlp_body(x_ref, w0_ref, b0_ref, w1_ref, b1_ref, wo_ref, bo_ref,
              sig_ref, geo_ref):
    x = x_ref[...]
    h = jnp.maximum(
        lax.dot_general(w0_ref[...], x, (((1,), (0,)), ((), ())),
                        preferred_element_type=jnp.float32)
        + b0_ref[...], 0.0)
    h = jnp.maximum(
        lax.dot_general(w1_ref[...], h, (((1,), (0,)), ((), ())),
                        preferred_element_type=jnp.float32)
        + b1_ref[...], 0.0)
    o = (lax.dot_general(wo_ref[...], h, (((1,), (0,)), ((), ())),
                         preferred_element_type=jnp.float32)
         + bo_ref[...])
    sig_ref[...] = jnp.exp(jnp.clip(o[0:1, :], -15.0, 15.0))
    geo_ref[...] = o[1:, :]


@jax.jit
def _tc_mlp(encT, w0t, b0c, w1t, b1c, wot, boc):
    n = encT.shape[1]
    grid = (n // BM,)
    full = lambda shape: pl.BlockSpec(shape, lambda i: (0, 0))
    sig, geoT = pl.pallas_call(
        _mlp_body,
        grid=grid,
        in_specs=[
            pl.BlockSpec((NUM_LEVELS * LEVEL_DIM, BM), lambda i: (0, i)),
            full((W_MLP, NUM_LEVELS * LEVEL_DIM)),
            full((W_MLP, 1)),
            full((W_MLP, W_MLP)),
            full((W_MLP, 1)),
            full((OUT_CH, W_MLP)),
            full((OUT_CH, 1)),
        ],
        out_specs=[
            pl.BlockSpec((1, BM), lambda i: (0, i)),
            pl.BlockSpec((N_GEO, BM), lambda i: (0, i)),
        ],
        out_shape=[
            jax.ShapeDtypeStruct((1, n), jnp.float32),
            jax.ShapeDtypeStruct((N_GEO, n), jnp.float32),
        ],
    )(encT, w0t, b0c, w1t, b1c, wot, boc)
    return sig, geoT


def kernel(xyzs, table, W0, b0, W1, b1, Wout, bout):
    n = xyzs.shape[0]
    xt = xyzs.T
    encT = _sc_encode(xt[0], xt[1], xt[2], table, n)
    sig, geoT = _tc_mlp(encT, W0.T, b0[:, None], W1.T, b1[:, None],
                        Wout.T, bout[:, None])
    return (sig[0], geoT.T)


# speedup vs baseline: 1.1941x; 1.1941x over previous
import functools

import jax
import jax.numpy as jnp
import numpy as np
from jax import lax
from jax.experimental import pallas as pl
from jax.experimental.pallas import tpu as pltpu
from jax.experimental.pallas import tpu_sc as plsc

BOUND = 1.0
NUM_LEVELS = 16
LEVEL_DIM = 2
BASE_RES = 16
LOG2_HASH = 19
MAX_RES = 2048
W_MLP = 64
N_GEO = 15
OUT_CH = 1 + N_GEO
PRIME1 = np.int32(np.int64(2654435761) - (1 << 32))
PRIME2 = np.int32(805459861)
HASH_MASK = np.int32((1 << LOG2_HASH) - 1)


def _levels():
    g = np.exp((np.log(MAX_RES) - np.log(BASE_RES)) / (NUM_LEVELS - 1))
    out, off = [], 0
    for l in range(NUM_LEVELS):
        res = int(np.floor(BASE_RES * (g**l)))
        size = min((res + 1) ** 3, 2**LOG2_HASH)
        size = int(np.ceil(size / 8) * 8)
        dense = (res + 1) ** 3 <= size
        out.append(dict(res=res, size=size, off=off, dense=dense))
        off += size
    return out, off


LEVELS, TOTAL_ROWS = _levels()

NUM_CORES = 2
NUM_SUBCORES = 16
NW = NUM_CORES * NUM_SUBCORES
LANES = 16

C = 512
NBLK = C // LANES
GW = 8


def _sc_encode_body(xs, ys, zs, table, enc_hbm,
                    x01x, x01y, x01z, fx, fy, fz, idx_v, rl_v, vals_v, enc_v,
                    sem, n_points):
    pw = n_points // NW
    nchunks = pw // C
    wid = lax.axis_index("s") * NUM_CORES + lax.axis_index("c")
    wbase = wid * pw
    lanes = lax.iota(jnp.int32, LANES)

    def chunk_body(k, _):
        pb = wbase + k * C
        pltpu.sync_copy(xs.at[pl.ds(pb, C)], x01x)
        pltpu.sync_copy(ys.at[pl.ds(pb, C)], x01y)
        pltpu.sync_copy(zs.at[pl.ds(pb, C)], x01z)

        def norm_body(i, _):
            i0 = i * LANES
            for ref in (x01x, x01y, x01z):
                v = ref[pl.ds(i0, LANES)]
                v = v * (0.5 / BOUND) + 0.5
                v = jnp.minimum(jnp.maximum(v, 0.0), 1.0)
                ref[pl.ds(i0, LANES)] = v
            return 0

        lax.fori_loop(0, NBLK, norm_body, 0)

        for li, lv in enumerate(LEVELS):
            scale = np.float32(lv["res"] - 1)
            R = np.int32(lv["res"] + 1)
            off = np.int32(lv["off"])

            def passA(i, _, scale=scale, R=R, off=off, dense=lv["dense"]):
                i0 = i * LANES
                px = x01x[pl.ds(i0, LANES)] * scale
                py = x01y[pl.ds(i0, LANES)] * scale
                pz = x01z[pl.ds(i0, LANES)] * scale
                ix0 = px.astype(jnp.int32)
                iy0 = py.astype(jnp.int32)
                iz0 = pz.astype(jnp.int32)
                fx[pl.ds(i0, LANES)] = px - ix0.astype(jnp.float32)
                fy[pl.ds(i0, LANES)] = py - iy0.astype(jnp.float32)
                fz[pl.ds(i0, LANES)] = pz - iz0.astype(jnp.float32)
                ix1 = ix0 + 1
                iy1 = iy0 + 1
                iz1 = iz0 + 1
                if dense:
                    ya = iy0 * R
                    yb = iy1 * R
                    za = iz0 * (R * R) + off
                    zb = iz1 * (R * R) + off
                    for c in range(8):
                        xi = ix1 if (c & 1) else ix0
                        yi = yb if (c & 2) else ya
                        zi = zb if (c & 4) else za
                        r = xi + yi + zi
                        e = c * C + i0
                        idx_v[pl.ds(e, LANES)] = lax.shift_right_logical(r, 2)
                        rl_v[pl.ds(e, LANES)] = (r & 3) * 2
                else:
                    ya = iy0 * PRIME1
                    yb = iy1 * PRIME1
                    za = iz0 * PRIME2
                    zb = iz1 * PRIME2
                    for c in range(8):
                        xi = ix1 if (c & 1) else ix0
                        yi = yb if (c & 2) else ya
                        zi = zb if (c & 4) else za
                        r = ((xi ^ yi ^ zi) & HASH_MASK) + off
                        e = c * C + i0
                        idx_v[pl.ds(e, LANES)] = lax.shift_right_logical(r, 2)
                        rl_v[pl.ds(e, LANES)] = (r & 3) * 2
                return 0

            lax.fori_loop(0, NBLK, passA, 0)

            pltpu.async_copy(table.at[idx_v], vals_v, sem).wait()

            def passB(i, _, li=li):
                i0 = i * LANES
                fxv = fx[pl.ds(i0, LANES)]
                fyv = fy[pl.ds(i0, LANES)]
                fzv = fz[pl.ds(i0, LANES)]
                gx = 1.0 - fxv
                gy = 1.0 - fyv
                gz = 1.0 - fzv
                rowb = i0 + lanes
                acc0 = jnp.zeros((LANES,), jnp.float32)
                acc1 = jnp.zeros((LANES,), jnp.float32)
                for c in range(8):
                    wx = fxv if (c & 1) else gx
                    wy = fyv if (c & 2) else gy
                    wz = fzv if (c & 4) else gz
                    w = (wx * wy) * wz
                    rows = rowb + np.int32(c * C)
                    rl = rl_v[pl.ds(c * C + i0, LANES)]
                    v0 = plsc.load_gather(vals_v, [rows, rl])
                    v1 = plsc.load_gather(vals_v, [rows, rl + 1])
                    acc0 = acc0 + v0 * w
                    acc1 = acc1 + v1 * w
                enc_v[2 * li, pl.ds(i0, LANES)] = acc0
                enc_v[2 * li + 1, pl.ds(i0, LANES)] = acc1
                return 0

            lax.fori_loop(0, NBLK, passB, 0)

        pltpu.sync_copy(enc_v, enc_hbm.at[:, pl.ds(pb, C)])
        return 0

    lax.fori_loop(0, nchunks, chunk_body, 0)


@functools.partial(jax.jit, static_argnames=("n_points",))
def _sc_encode(xs, ys, zs, table, n_points):
    mesh = plsc.VectorSubcoreMesh(
        core_axis_name="c", subcore_axis_name="s",
        num_cores=NUM_CORES, num_subcores=NUM_SUBCORES)
    body = functools.partial(_sc_encode_body, n_points=n_points)
    return pl.kernel(
        body,
        out_type=jax.ShapeDtypeStruct((NUM_LEVELS * LEVEL_DIM, n_points),
                                      jnp.float32),
        mesh=mesh,
        compiler_params=pltpu.CompilerParams(
            needs_layout_passes=False, use_tc_tiling_on_sc=False),
        scratch_types=[
            pltpu.VMEM((C,), jnp.float32),
            pltpu.VMEM((C,), jnp.float32),
            pltpu.VMEM((C,), jnp.float32),
            pltpu.VMEM((C,), jnp.float32),
            pltpu.VMEM((C,), jnp.float32),
            pltpu.VMEM((C,), jnp.float32),
            pltpu.VMEM((8 * C,), jnp.int32),
            pltpu.VMEM((8 * C,), jnp.int32),
            pltpu.VMEM((8 * C, GW), jnp.float32),
            pltpu.VMEM((NUM_LEVELS * LEVEL_DIM, C), jnp.float32),
            pltpu.SemaphoreType.DMA,
        ],
    )(xs, ys, zs, table)


BM = 4096


def _mlp_body(x_ref, w0_ref, b0_ref, w1_ref, b1_ref, wo_ref, bo_ref,
              sig_ref, geo_ref):
    x = x_ref[...]
    h = jnp.maximum(
        lax.dot_general(w0_ref[...], x, (((1,), (0,)), ((), ())),
                        preferred_element_type=jnp.float32)
        + b0_ref[...], 0.0)
    h = jnp.maximum(
        lax.dot_general(w1_ref[...], h, (((1,), (0,)), ((), ())),
                        preferred_element_type=jnp.float32)
        + b1_ref[...], 0.0)
    o = (lax.dot_general(wo_ref[...], h, (((1,), (0,)), ((), ())),
                         preferred_element_type=jnp.float32)
         + bo_ref[...])
    sig_ref[...] = jnp.exp(jnp.clip(o[0:1, :], -15.0, 15.0))
    geo_ref[...] = o[1:, :]


@jax.jit
def _tc_mlp(encT, w0t, b0c, w1t, b1c, wot, boc):
    n = encT.shape[1]
    grid = (n // BM,)
    full = lambda shape: pl.BlockSpec(shape, lambda i: (0, 0))
    sig, geoT = pl.pallas_call(
        _mlp_body,
        grid=grid,
        in_specs=[
            pl.BlockSpec((NUM_LEVELS * LEVEL_DIM, BM), lambda i: (0, i)),
            full((W_MLP, NUM_LEVELS * LEVEL_DIM)),
            full((W_MLP, 1)),
            full((W_MLP, W_MLP)),
            full((W_MLP, 1)),
            full((OUT_CH, W_MLP)),
            full((OUT_CH, 1)),
        ],
        out_specs=[
            pl.BlockSpec((1, BM), lambda i: (0, i)),
            pl.BlockSpec((N_GEO, BM), lambda i: (0, i)),
        ],
        out_shape=[
            jax.ShapeDtypeStruct((1, n), jnp.float32),
            jax.ShapeDtypeStruct((N_GEO, n), jnp.float32),
        ],
    )(encT, w0t, b0c, w1t, b1c, wot, boc)
    return sig, geoT


def kernel(xyzs, table, W0, b0, W1, b1, Wout, bout):
    n = xyzs.shape[0]
    xt = xyzs.T
    table8 = table.reshape(-1, GW)
    encT = _sc_encode(xt[0], xt[1], xt[2], table8, n)
    sig, geoT = _tc_mlp(encT, W0.T, b0[:, None], W1.T, b1[:, None],
                        Wout.T, bout[:, None])
    return (sig[0], geoT.T)
